# Initial kernel scaffold; baseline (speedup 1.0000x reference)
import jax
import jax.numpy as jnp
from jax.experimental import pallas as pl


def kernel(flat, cu_seqlens):
    raise NotImplementedError("write your pallas kernel here")


# speedup vs baseline: 8.4381x; 8.4381x over previous
import functools

import jax
import jax.numpy as jnp
from jax import lax
from jax.experimental import pallas as pl
from jax.experimental.pallas import tpu as pltpu
from jax.experimental.pallas import tpu_sc as plsc

VOCAB_NUM = 100000
SEQ = 4096
BATCH = 16
TOTAL = 32768
HALF = SEQ // 2
LANES = 16
CHUNKS = HALF // LANES
WIN = HALF + 8
PAD_LEN = TOTAL + SEQ // 2 + WIN


def _body(flat_hbm, cu_hbm, out_hbm, cu_v, buf_v, row_v):
    c = lax.axis_index("c")
    s = lax.axis_index("s")
    wid = c * 16 + s
    b = wid // 2
    h = wid % 2

    pltpu.sync_copy(cu_hbm, cu_v)

    v = cu_v[pl.ds(b, LANES)]
    start = v[0]
    end = v[1]
    seg_len = jnp.minimum(end - start, SEQ)

    src0 = start + h * HALF
    base = pl.multiple_of(jnp.bitwise_and(src0, jnp.int32(-8)), 8)
    rem = src0 - base

    pltpu.sync_copy(flat_hbm.at[pl.ds(base, WIN)], buf_v)

    col0 = h * HALF
    lanes = lax.iota(jnp.int32, LANES)

    def step(k, _):
        chunk = buf_v[pl.ds(rem + k * LANES, LANES)]
        hashed = lax.rem(chunk, jnp.int32(VOCAB_NUM))
        col = col0 + k * LANES + lanes
        row_v[pl.ds(k * LANES, LANES)] = jnp.where(col < seg_len, hashed, 0)
        return _

    lax.fori_loop(0, CHUNKS, step, None)

    pltpu.sync_copy(row_v, out_hbm.at[pl.ds(b * SEQ + col0, HALF)])


@functools.cache
def _get_packer():
    mesh = plsc.VectorSubcoreMesh(core_axis_name="c", subcore_axis_name="s")
    return functools.partial(
        pl.kernel,
        out_type=jax.ShapeDtypeStruct((BATCH * SEQ,), jnp.int32),
        mesh=mesh,
        scratch_types=[
            pltpu.VMEM((2 * LANES,), jnp.int32),
            pltpu.VMEM((WIN,), jnp.int32),
            pltpu.VMEM((HALF,), jnp.int32),
        ],
    )(_body)


def kernel(flat, cu_seqlens):
    out_dtype = flat.dtype
    flat_i = flat.astype(jnp.int32)
    flat_p = jnp.pad(flat_i, (0, PAD_LEN - TOTAL))
    cu_pad = jnp.pad(cu_seqlens.astype(jnp.int32),
                     (0, 2 * LANES - cu_seqlens.shape[0]))
    out = _get_packer()(flat_p, cu_pad)
    return out.reshape(BATCH, SEQ).astype(out_dtype)


# speedup vs baseline: 10.4717x; 1.2410x over previous
import functools

import jax
import jax.numpy as jnp
from jax import lax
from jax.experimental import pallas as pl
from jax.experimental.pallas import tpu as pltpu
from jax.experimental.pallas import tpu_sc as plsc

VOCAB_NUM = 100000
SEQ = 4096
BATCH = 16
TOTAL = 32768
HALF = SEQ // 2
LANES = 16
CHUNKS = HALF // LANES
WIN = HALF + 8
PAD_LEN = TOTAL + SEQ // 2 + WIN


def _body(flat_hbm, cu_hbm, out_hbm, cu_v, buf_v, row_v):
    c = lax.axis_index("c")
    s = lax.axis_index("s")
    wid = c * 16 + s
    b = wid // 2
    h = wid % 2

    pltpu.sync_copy(cu_hbm, cu_v)

    v = cu_v[pl.ds(b, LANES)]
    start = v[0]
    end = v[1]
    seg_len = jnp.minimum(end - start, SEQ)

    src0 = start + h * HALF
    base = pl.multiple_of(jnp.bitwise_and(src0, jnp.int32(-8)), 8)
    rem = src0 - base

    pltpu.sync_copy(flat_hbm.at[pl.ds(base, WIN)], buf_v)

    col0 = h * HALF
    lanes = lax.iota(jnp.int32, LANES)

    n_valid = jnp.clip(seg_len - col0, 0, HALF)
    k_valid = (n_valid + LANES - 1) // LANES

    @functools.partial(plsc.parallel_loop, 0, k_valid, unroll=8)
    def _compute(k):
        chunk = buf_v[pl.ds(rem + k * LANES, LANES)]
        hashed = lax.rem(chunk, jnp.int32(VOCAB_NUM))
        col = col0 + k * LANES + lanes
        row_v[pl.ds(k * LANES, LANES)] = jnp.where(col < seg_len, hashed, 0)

    zero = jnp.zeros((LANES,), jnp.int32)

    @functools.partial(plsc.parallel_loop, k_valid, CHUNKS, unroll=8)
    def _pad(k):
        row_v[pl.ds(k * LANES, LANES)] = zero

    pltpu.sync_copy(row_v, out_hbm.at[pl.ds(b * SEQ + col0, HALF)])


@functools.cache
def _get_packer():
    mesh = plsc.VectorSubcoreMesh(core_axis_name="c", subcore_axis_name="s")
    return functools.partial(
        pl.kernel,
        out_type=jax.ShapeDtypeStruct((BATCH * SEQ,), jnp.int32),
        mesh=mesh,
        scratch_types=[
            pltpu.VMEM((2 * LANES,), jnp.int32),
            pltpu.VMEM((WIN,), jnp.int32),
            pltpu.VMEM((HALF,), jnp.int32),
        ],
    )(_body)


def kernel(flat, cu_seqlens):
    out_dtype = flat.dtype
    flat_i = flat.astype(jnp.int32)
    flat_p = jnp.pad(flat_i, (0, PAD_LEN - TOTAL))
    cu_pad = jnp.pad(cu_seqlens.astype(jnp.int32),
                     (0, 2 * LANES - cu_seqlens.shape[0]))
    out = _get_packer()(flat_p, cu_pad)
    return out.reshape(BATCH, SEQ).astype(out_dtype)
